# baseline (speedup 1.0000x reference)
import jax
import jax.numpy as jnp
from jax import lax
from jax.experimental import pallas as pl
from jax.experimental.pallas import tpu as pltpu
from jax.experimental.pallas import tpu_sc as plsc

VOCAB = 16384
HIDDEN = 1024
C = 16
S = VOCAB // C
N = 4096
B = 128
NBLK = N // B + C
NPAD = NBLK * B
L = 16
NSUB = 16
TPW = N // NSUB
SPAD = 8
GW = 32
RPW = NPAD // GW
NCH = 6
GCH = RPW // NCH
NBUF = 3


def _lane():
    return lax.broadcasted_iota(jnp.int32, (L,), 0)


_GDN = lax.GatherDimensionNumbers(
    offset_dims=(), collapsed_slice_dims=(0,), start_index_map=(0,))


def _vperm(vals, idx):
    return lax.gather(vals, idx[:, None], _GDN, (1,),
                      mode=lax.GatherScatterMode.PROMISE_IN_BOUNDS)


HW = HIDDEN // 2


def _pack_body(x_ref, o_ref):
    u = pltpu.bitcast(x_ref[...], jnp.uint32)
    r = (u + jnp.uint32(0x7FFF) + ((u >> 16) & jnp.uint32(1))) >> 16
    packed = (r[:, HW:] << 16) | r[:, :HW]
    o_ref[...] = pltpu.bitcast(packed, jnp.int32)


def _pack_x(x):
    return pl.pallas_call(
        _pack_body,
        grid=(8,),
        in_specs=[pl.BlockSpec((N // 8, HIDDEN), lambda i: (i, 0))],
        out_specs=pl.BlockSpec((N // 8, HW), lambda i: (i, 0)),
        out_shape=jax.ShapeDtypeStruct((N, HW), jnp.int32),
    )(x)


def _route_body(ypos_hbm, y_hbm, slot_hbm, perm_hbm, tcol_hbm, bc_hbm,
                yp_v, y_v, hist_v, cur_v, all_v, slot_v, sall_v, tall_v,
                call_v, sidx0_v, sidx1_v, tok0_v, tok1_v, tc0_v, tc1_v,
                bc_v, shared_hist, sem):
    w = lax.axis_index("s")
    t0 = w * TPW
    lane = _lane()
    ones = jnp.ones((L,), jnp.int32)
    pltpu.sync_copy(ypos_hbm.at[pl.ds(t0, TPW)], yp_v)
    pltpu.sync_copy(y_hbm.at[pl.ds(t0, TPW)], y_v)

    hist_v[...] = jnp.zeros((L,), jnp.int32)

    def hist_step(k, _):
        v = yp_v[pl.ds(k * L, L)]
        plsc.addupdate_scatter(hist_v, [v], ones)
        return 0

    lax.fori_loop(0, TPW // L, hist_step, 0)
    pltpu.sync_copy(hist_v, shared_hist.at[w + SPAD])
    plsc.subcore_barrier()
    plsc.subcore_barrier()
    pltpu.sync_copy(shared_hist.at[pl.ds(SPAD, NSUB)], all_v)

    prefix = jnp.zeros((L,), jnp.int32)
    total = jnp.zeros((L,), jnp.int32)
    for w2 in range(NSUB):
        h = all_v[w2]
        prefix = prefix + h * jnp.where(w2 < w, 1, 0)
        total = total + h
    nblk = (total + (B - 1)) >> 7
    blkstart = plsc.cumsum(nblk) - nblk
    base = (blkstart << 7) + prefix

    cur_v[...] = base

    def slot_step(k, _):
        v = yp_v[pl.ds(k * L, L)]
        yv = y_v[pl.ds(k * L, L)]
        rank = jnp.zeros((L,), jnp.int32)
        for sft in range(1, L):
            sh = _vperm(v, jnp.maximum(lane - sft, 0))
            rank = rank + jnp.where((lane >= sft) & (sh == v), 1, 0)
        slotv = _vperm(cur_v[...], v) + rank
        plsc.addupdate_scatter(cur_v, [v], ones)
        slot_v[pl.ds(k * L, L)] = slotv
        sall_v[pl.ds(k * L, L)] = slotv
        tall_v[pl.ds(k * L, L)] = t0 + k * L + lane
        call_v[pl.ds(k * L, L)] = yv & (S - 1)
        return 0

    lax.fori_loop(0, TPW // L, slot_step, 0)

    for q in range(8):
        o = q * L
        sidx0_v[pl.ds(o, L)] = sall_v[pl.ds(o, L)]
        tok0_v[pl.ds(o, L)] = tall_v[pl.ds(o, L)]
        tc0_v[pl.ds(o, L)] = call_v[pl.ds(o, L)]
        sidx1_v[pl.ds(o, L)] = sall_v[pl.ds(128 + o, L)]
        tok1_v[pl.ds(o, L)] = tall_v[pl.ds(128 + o, L)]
        tc1_v[pl.ds(o, L)] = call_v[pl.ds(128 + o, L)]

    pltpu.sync_copy(slot_v, slot_hbm.at[pl.ds(t0, TPW)])
    d0 = pltpu.async_copy(tok0_v, perm_hbm.at[sidx0_v], sem)
    d1 = pltpu.async_copy(tok1_v, perm_hbm.at[sidx1_v], sem)
    d2 = pltpu.async_copy(tc0_v, tcol_hbm.at[sidx0_v], sem)
    d3 = pltpu.async_copy(tc1_v, tcol_hbm.at[sidx1_v], sem)
    d0.wait()
    d1.wait()
    d2.wait()
    d3.wait()

    @pl.when(w == 0)
    def _():
        for r in range(NBLK // L):
            bi = lane + r * L
            cnt = jnp.zeros((L,), jnp.int32)
            for b in range(C):
                st = _vperm(blkstart, jnp.zeros((L,), jnp.int32) + b)
                cnt = cnt + jnp.where(st <= bi, 1, 0)
            bc_v[pl.ds(r * L, L)] = jnp.minimum(cnt - 1, C - 1)
        pltpu.sync_copy(bc_v, bc_hbm)


def _route(y_pos, y):
    mesh = plsc.VectorSubcoreMesh(
        core_axis_name="c", subcore_axis_name="s", num_cores=1)
    f = pl.kernel(
        _route_body,
        compiler_params=pltpu.CompilerParams(needs_layout_passes=False),
        out_type=(
            jax.ShapeDtypeStruct((N,), jnp.int32),
            jax.ShapeDtypeStruct((NPAD,), jnp.int32),
            jax.ShapeDtypeStruct((NPAD,), jnp.int32),
            jax.ShapeDtypeStruct((NBLK,), jnp.int32),
        ),
        mesh=mesh,
        scratch_types=[
            pltpu.VMEM((TPW,), jnp.int32),
            pltpu.VMEM((TPW,), jnp.int32),
            pltpu.VMEM((L,), jnp.int32),
            pltpu.VMEM((L,), jnp.int32),
            pltpu.VMEM((NSUB, L), jnp.int32),
            pltpu.VMEM((TPW,), jnp.int32),
            pltpu.VMEM((TPW,), jnp.int32),
            pltpu.VMEM((TPW,), jnp.int32),
            pltpu.VMEM((TPW,), jnp.int32),
            pltpu.VMEM((128,), jnp.int32),
            pltpu.VMEM((128,), jnp.int32),
            pltpu.VMEM((128,), jnp.int32),
            pltpu.VMEM((128,), jnp.int32),
            pltpu.VMEM((128,), jnp.int32),
            pltpu.VMEM((128,), jnp.int32),
            pltpu.VMEM((NBLK,), jnp.int32),
            pltpu.VMEM_SHARED((NSUB + SPAD, L), jnp.int32),
            pltpu.SemaphoreType.DMA,
        ],
    )
    return f(y_pos, y)


def _gather_body(x_hbm, perm_hbm, xs_hbm, idx_v, r0, r1, r2, s0):
    wid = lax.axis_index("s") * 2 + lax.axis_index("c")
    base = wid * RPW
    for j in range(NCH):
        pltpu.sync_copy(perm_hbm.at[pl.ds(base + j * GCH, GCH)], idx_v.at[j])
    for q in range(RPW // L):
        r, o = q // (GCH // L), (q % (GCH // L)) * L
        iv = idx_v[r, pl.ds(o, L)]
        idx_v[r, pl.ds(o, L)] = jnp.minimum(jnp.maximum(iv, 0), N - 1)
    rows = (r0, r1, r2)
    for ph in range(NCH // NBUF):
        pend = []
        for j in range(NBUF):
            pend.append(pltpu.async_copy(
                x_hbm.at[idx_v.at[ph * NBUF + j]], rows[j], s0))
        for j in range(NBUF):
            pend[j].wait()
        for j in range(NBUF):
            pltpu.sync_copy(
                rows[j], xs_hbm.at[pl.ds(base + (ph * NBUF + j) * GCH, GCH)])


def _gather_rows(x, perm):
    mesh = plsc.VectorSubcoreMesh(core_axis_name="c", subcore_axis_name="s")
    f = pl.kernel(
        _gather_body,
        out_type=jax.ShapeDtypeStruct((NPAD, HW), jnp.int32),
        mesh=mesh,
        scratch_types=[pltpu.VMEM((NCH, GCH), jnp.int32)]
        + [pltpu.VMEM((GCH, HW), jnp.int32)] * NBUF
        + [pltpu.SemaphoreType.DMA],
    )
    return f(x, perm)


def _mm_body(bc_ref, xs_ref, w_ref, wc_ref, tcol_ref, out_ref,
             wb_ref, wcb_ref):
    i = pl.program_id(0)
    c = bc_ref[i]
    prev = bc_ref[jnp.maximum(i - 1, 0)]

    @pl.when(i == 0)
    def _():
        wcb_ref[...] = wc_ref[...].astype(jnp.bfloat16)

    @pl.when((i == 0) | (c != prev))
    def _():
        wb_ref[...] = w_ref[...].astype(jnp.bfloat16)

    ub = pltpu.bitcast(xs_ref[...], jnp.uint32)
    lo = pltpu.bitcast(ub << 16, jnp.float32)
    hi = pltpu.bitcast(ub & jnp.uint32(0xFFFF0000), jnp.float32)
    xb = jnp.concatenate([lo, hi], axis=1).astype(jnp.bfloat16)
    t = jnp.dot(xb, wb_ref[...],
                preferred_element_type=jnp.float32)
    onev = jnp.ones((S, 1), jnp.bfloat16)
    se = jnp.dot(jnp.exp(t).astype(jnp.bfloat16), onev,
                 preferred_element_type=jnp.float32)
    lse = jnp.log(se)
    tcol = tcol_ref[0]
    masked = jnp.where(
        lax.broadcasted_iota(jnp.int32, (B, S), 1) == tcol, t, 0.0)
    tgt = jnp.dot(masked.astype(jnp.bfloat16), onev,
                  preferred_element_type=jnp.float32)

    cl = lax.dot_general(xb, wcb_ref[...], (((1,), (1,)), ((), ())),
                         preferred_element_type=jnp.float32)
    lse_c = jnp.log(jnp.sum(jnp.exp(cl), axis=1, keepdims=True))
    sel = jnp.sum(jnp.where(
        lax.broadcasted_iota(jnp.int32, (B, C), 1) == c, cl, 0.0),
        axis=1, keepdims=True)

    out_ref[0] = (lse_c - sel) + (lse - tgt)


def _tail_nll(bc, xs, logits, W_cluster, tcol):
    grid_spec = pltpu.PrefetchScalarGridSpec(
        num_scalar_prefetch=1,
        grid=(NBLK,),
        in_specs=[
            pl.BlockSpec((B, HW), lambda i, bc: (i, 0)),
            pl.BlockSpec((HIDDEN, S), lambda i, bc: (0, bc[i])),
            pl.BlockSpec((C, HIDDEN), lambda i, bc: (0, 0)),
            pl.BlockSpec((1, B, 1), lambda i, bc: (i, 0, 0)),
        ],
        out_specs=pl.BlockSpec((1, B, 1), lambda i, bc: (i, 0, 0)),
        scratch_shapes=[
            pltpu.VMEM((HIDDEN, S), jnp.bfloat16),
            pltpu.VMEM((C, HIDDEN), jnp.bfloat16),
        ],
    )
    return pl.pallas_call(
        _mm_body,
        grid_spec=grid_spec,
        out_shape=jax.ShapeDtypeStruct((NBLK, B, 1), jnp.float32),
    )(bc, xs, logits, W_cluster, tcol.reshape(NBLK, B, 1))


def _final_body(slot_hbm, nlls_hbm, out_hbm, sidx_v, vals_v, sem):
    wid = lax.axis_index("s") * 2 + lax.axis_index("c")
    base = wid * (N // GW)
    pltpu.sync_copy(slot_hbm.at[pl.ds(base, N // GW)], sidx_v)
    for q in range((N // GW) // L):
        iv = sidx_v[pl.ds(q * L, L)]
        sidx_v[pl.ds(q * L, L)] = jnp.minimum(jnp.maximum(iv, 0), NPAD - 1)
    pltpu.async_copy(nlls_hbm.at[sidx_v], vals_v, sem).wait()
    pltpu.sync_copy(vals_v, out_hbm.at[pl.ds(base, N // GW)])


def _finalize(slot, nlls):
    mesh = plsc.VectorSubcoreMesh(core_axis_name="c", subcore_axis_name="s")
    f = pl.kernel(
        _final_body,
        out_type=jax.ShapeDtypeStruct((N,), jnp.float32),
        mesh=mesh,
        scratch_types=[
            pltpu.VMEM((N // GW,), jnp.int32),
            pltpu.VMEM((N // GW,), jnp.float32),
            pltpu.SemaphoreType.DMA,
        ],
    )
    return f(slot, nlls)


def kernel(x, y, y_pos, W_cluster, logits):
    x2 = _pack_x(x)
    slot, perm, tcol, bc = _route(y_pos, y)
    xs2 = _gather_rows(x2, perm)
    nlls = _tail_nll(bc, xs2, logits, W_cluster, tcol)
    return _finalize(slot, nlls.reshape(NPAD))


# speedup vs baseline: 1.1220x; 1.1220x over previous
import jax
import jax.numpy as jnp
from jax import lax
from jax.experimental import pallas as pl
from jax.experimental.pallas import tpu as pltpu
from jax.experimental.pallas import tpu_sc as plsc

VOCAB = 16384
HIDDEN = 1024
C = 16
S = VOCAB // C
N = 4096
B = 64
BSH = 6
NBLK = N // B + C
NPAD = NBLK * B
L = 16
NSUB = 16
TPW = N // NSUB
SPAD = 8
GW = 32
RPW = NPAD // GW
NCH = 5
GCH = RPW // NCH
NBUF = 3


def _lane():
    return lax.broadcasted_iota(jnp.int32, (L,), 0)


_GDN = lax.GatherDimensionNumbers(
    offset_dims=(), collapsed_slice_dims=(0,), start_index_map=(0,))


def _vperm(vals, idx):
    return lax.gather(vals, idx[:, None], _GDN, (1,),
                      mode=lax.GatherScatterMode.PROMISE_IN_BOUNDS)


HW = HIDDEN // 2


def _pack_body(x_ref, o_ref):
    u = pltpu.bitcast(x_ref[...], jnp.uint32)
    r = (u + jnp.uint32(0x7FFF) + ((u >> 16) & jnp.uint32(1))) >> 16
    packed = (r[:, HW:] << 16) | r[:, :HW]
    o_ref[...] = pltpu.bitcast(packed, jnp.int32)


def _pack_x(x):
    return pl.pallas_call(
        _pack_body,
        grid=(8,),
        in_specs=[pl.BlockSpec((N // 8, HIDDEN), lambda i: (i, 0))],
        out_specs=pl.BlockSpec((N // 8, HW), lambda i: (i, 0)),
        out_shape=jax.ShapeDtypeStruct((N, HW), jnp.int32),
    )(x)


def _route_body(ypos_hbm, y_hbm, slot_hbm, perm_hbm, tcol_hbm, bc_hbm,
                yp_v, y_v, hist_v, cur_v, all_v, slot_v, sall_v, tall_v,
                call_v, sidx0_v, sidx1_v, tok0_v, tok1_v, tc0_v, tc1_v,
                bc_v, shared_hist, sem):
    w = lax.axis_index("s")
    t0 = w * TPW
    lane = _lane()
    ones = jnp.ones((L,), jnp.int32)
    pltpu.sync_copy(ypos_hbm.at[pl.ds(t0, TPW)], yp_v)
    pltpu.sync_copy(y_hbm.at[pl.ds(t0, TPW)], y_v)

    hist_v[...] = jnp.zeros((L,), jnp.int32)

    def hist_step(k, _):
        v = yp_v[pl.ds(k * L, L)]
        plsc.addupdate_scatter(hist_v, [v], ones)
        return 0

    lax.fori_loop(0, TPW // L, hist_step, 0)
    pltpu.sync_copy(hist_v, shared_hist.at[w + SPAD])
    plsc.subcore_barrier()
    plsc.subcore_barrier()
    pltpu.sync_copy(shared_hist.at[pl.ds(SPAD, NSUB)], all_v)

    prefix = jnp.zeros((L,), jnp.int32)
    total = jnp.zeros((L,), jnp.int32)
    for w2 in range(NSUB):
        h = all_v[w2]
        prefix = prefix + h * jnp.where(w2 < w, 1, 0)
        total = total + h
    nblk = (total + (B - 1)) >> BSH
    blkstart = plsc.cumsum(nblk) - nblk
    base = (blkstart << BSH) + prefix

    cur_v[...] = base

    def slot_step(k, _):
        v = yp_v[pl.ds(k * L, L)]
        yv = y_v[pl.ds(k * L, L)]
        rank = jnp.zeros((L,), jnp.int32)
        for sft in range(1, L):
            sh = _vperm(v, jnp.maximum(lane - sft, 0))
            rank = rank + jnp.where((lane >= sft) & (sh == v), 1, 0)
        slotv = _vperm(cur_v[...], v) + rank
        plsc.addupdate_scatter(cur_v, [v], ones)
        slot_v[pl.ds(k * L, L)] = slotv
        sall_v[pl.ds(k * L, L)] = slotv
        tall_v[pl.ds(k * L, L)] = t0 + k * L + lane
        call_v[pl.ds(k * L, L)] = yv & (S - 1)
        return 0

    lax.fori_loop(0, TPW // L, slot_step, 0)

    for q in range(8):
        o = q * L
        sidx0_v[pl.ds(o, L)] = sall_v[pl.ds(o, L)]
        tok0_v[pl.ds(o, L)] = tall_v[pl.ds(o, L)]
        tc0_v[pl.ds(o, L)] = call_v[pl.ds(o, L)]
        sidx1_v[pl.ds(o, L)] = sall_v[pl.ds(128 + o, L)]
        tok1_v[pl.ds(o, L)] = tall_v[pl.ds(128 + o, L)]
        tc1_v[pl.ds(o, L)] = call_v[pl.ds(128 + o, L)]

    pltpu.sync_copy(slot_v, slot_hbm.at[pl.ds(t0, TPW)])
    d0 = pltpu.async_copy(tok0_v, perm_hbm.at[sidx0_v], sem)
    d1 = pltpu.async_copy(tok1_v, perm_hbm.at[sidx1_v], sem)
    d2 = pltpu.async_copy(tc0_v, tcol_hbm.at[sidx0_v], sem)
    d3 = pltpu.async_copy(tc1_v, tcol_hbm.at[sidx1_v], sem)
    d0.wait()
    d1.wait()
    d2.wait()
    d3.wait()

    @pl.when(w == 0)
    def _():
        for r in range(NBLK // L):
            bi = lane + r * L
            cnt = jnp.zeros((L,), jnp.int32)
            for b in range(C):
                st = _vperm(blkstart, jnp.zeros((L,), jnp.int32) + b)
                cnt = cnt + jnp.where(st <= bi, 1, 0)
            bc_v[pl.ds(r * L, L)] = jnp.minimum(cnt - 1, C - 1)
        pltpu.sync_copy(bc_v, bc_hbm)


def _route(y_pos, y):
    mesh = plsc.VectorSubcoreMesh(
        core_axis_name="c", subcore_axis_name="s", num_cores=1)
    f = pl.kernel(
        _route_body,
        compiler_params=pltpu.CompilerParams(needs_layout_passes=False),
        out_type=(
            jax.ShapeDtypeStruct((N,), jnp.int32),
            jax.ShapeDtypeStruct((NPAD,), jnp.int32),
            jax.ShapeDtypeStruct((NPAD,), jnp.int32),
            jax.ShapeDtypeStruct((NBLK,), jnp.int32),
        ),
        mesh=mesh,
        scratch_types=[
            pltpu.VMEM((TPW,), jnp.int32),
            pltpu.VMEM((TPW,), jnp.int32),
            pltpu.VMEM((L,), jnp.int32),
            pltpu.VMEM((L,), jnp.int32),
            pltpu.VMEM((NSUB, L), jnp.int32),
            pltpu.VMEM((TPW,), jnp.int32),
            pltpu.VMEM((TPW,), jnp.int32),
            pltpu.VMEM((TPW,), jnp.int32),
            pltpu.VMEM((TPW,), jnp.int32),
            pltpu.VMEM((128,), jnp.int32),
            pltpu.VMEM((128,), jnp.int32),
            pltpu.VMEM((128,), jnp.int32),
            pltpu.VMEM((128,), jnp.int32),
            pltpu.VMEM((128,), jnp.int32),
            pltpu.VMEM((128,), jnp.int32),
            pltpu.VMEM((NBLK,), jnp.int32),
            pltpu.VMEM_SHARED((NSUB + SPAD, L), jnp.int32),
            pltpu.SemaphoreType.DMA,
        ],
    )
    return f(y_pos, y)


def _gather_body(x_hbm, perm_hbm, xs_hbm, idx_v, r0, r1, r2, s0):
    wid = lax.axis_index("s") * 2 + lax.axis_index("c")
    base = wid * RPW
    for j in range(NCH):
        pltpu.sync_copy(perm_hbm.at[pl.ds(base + j * GCH, GCH)], idx_v.at[j])
    for q in range(RPW // L):
        r, o = q // (GCH // L), (q % (GCH // L)) * L
        iv = idx_v[r, pl.ds(o, L)]
        idx_v[r, pl.ds(o, L)] = jnp.minimum(jnp.maximum(iv, 0), N - 1)
    rows = (r0, r1, r2)
    for ph in range(0, NCH, NBUF):
        k = min(NBUF, NCH - ph)
        pend = []
        for j in range(k):
            pend.append(pltpu.async_copy(
                x_hbm.at[idx_v.at[ph + j]], rows[j], s0))
        for j in range(k):
            pend[j].wait()
        for j in range(k):
            pltpu.sync_copy(
                rows[j], xs_hbm.at[pl.ds(base + (ph + j) * GCH, GCH)])


def _gather_rows(x, perm):
    mesh = plsc.VectorSubcoreMesh(core_axis_name="c", subcore_axis_name="s")
    f = pl.kernel(
        _gather_body,
        out_type=jax.ShapeDtypeStruct((NPAD, HW), jnp.int32),
        mesh=mesh,
        scratch_types=[pltpu.VMEM((NCH, GCH), jnp.int32)]
        + [pltpu.VMEM((GCH, HW), jnp.int32)] * NBUF
        + [pltpu.SemaphoreType.DMA],
    )
    return f(x, perm)


def _mm_body(bc_ref, xs_ref, w_ref, wc_ref, tcol_ref, out_ref,
             wb_ref, wcb_ref):
    i = pl.program_id(0)
    c = bc_ref[i]
    prev = bc_ref[jnp.maximum(i - 1, 0)]

    @pl.when(i == 0)
    def _():
        wcb_ref[...] = wc_ref[...].astype(jnp.bfloat16)

    @pl.when((i == 0) | (c != prev))
    def _():
        wb_ref[...] = w_ref[...].astype(jnp.bfloat16)

    ub = pltpu.bitcast(xs_ref[...], jnp.uint32)
    lo = pltpu.bitcast(ub << 16, jnp.float32)
    hi = pltpu.bitcast(ub & jnp.uint32(0xFFFF0000), jnp.float32)
    xb = jnp.concatenate([lo, hi], axis=1).astype(jnp.bfloat16)
    t = jnp.dot(xb, wb_ref[...],
                preferred_element_type=jnp.float32)
    lse = jnp.log(jnp.sum(jnp.exp(t), axis=1, keepdims=True))
    tcol = tcol_ref[0]
    tgt = jnp.sum(jnp.where(
        lax.broadcasted_iota(jnp.int32, (B, S), 1) == tcol, t, 0.0),
        axis=1, keepdims=True)

    cl = lax.dot_general(xb, wcb_ref[...], (((1,), (1,)), ((), ())),
                         preferred_element_type=jnp.float32)
    lse_c = jnp.log(jnp.sum(jnp.exp(cl), axis=1, keepdims=True))
    sel = jnp.sum(jnp.where(
        lax.broadcasted_iota(jnp.int32, (B, C), 1) == c, cl, 0.0),
        axis=1, keepdims=True)

    out_ref[0] = (lse_c - sel) + (lse - tgt)


def _tail_nll(bc, xs, logits, W_cluster, tcol):
    grid_spec = pltpu.PrefetchScalarGridSpec(
        num_scalar_prefetch=1,
        grid=(NBLK,),
        in_specs=[
            pl.BlockSpec((B, HW), lambda i, bc: (i, 0)),
            pl.BlockSpec((HIDDEN, S), lambda i, bc: (0, bc[i])),
            pl.BlockSpec((C, HIDDEN), lambda i, bc: (0, 0)),
            pl.BlockSpec((1, B, 1), lambda i, bc: (i, 0, 0)),
        ],
        out_specs=pl.BlockSpec((1, B, 1), lambda i, bc: (i, 0, 0)),
        scratch_shapes=[
            pltpu.VMEM((HIDDEN, S), jnp.bfloat16),
            pltpu.VMEM((C, HIDDEN), jnp.bfloat16),
        ],
    )
    return pl.pallas_call(
        _mm_body,
        grid_spec=grid_spec,
        out_shape=jax.ShapeDtypeStruct((NBLK, B, 1), jnp.float32),
    )(bc, xs, logits, W_cluster, tcol.reshape(NBLK, B, 1))


def _final_body(slot_hbm, nlls_hbm, out_hbm, sidx_v, vals_v, sem):
    wid = lax.axis_index("s") * 2 + lax.axis_index("c")
    base = wid * (N // GW)
    pltpu.sync_copy(slot_hbm.at[pl.ds(base, N // GW)], sidx_v)
    for q in range((N // GW) // L):
        iv = sidx_v[pl.ds(q * L, L)]
        sidx_v[pl.ds(q * L, L)] = jnp.minimum(jnp.maximum(iv, 0), NPAD - 1)
    pltpu.async_copy(nlls_hbm.at[sidx_v], vals_v, sem).wait()
    pltpu.sync_copy(vals_v, out_hbm.at[pl.ds(base, N // GW)])


def _finalize(slot, nlls):
    mesh = plsc.VectorSubcoreMesh(core_axis_name="c", subcore_axis_name="s")
    f = pl.kernel(
        _final_body,
        out_type=jax.ShapeDtypeStruct((N,), jnp.float32),
        mesh=mesh,
        scratch_types=[
            pltpu.VMEM((N // GW,), jnp.int32),
            pltpu.VMEM((N // GW,), jnp.float32),
            pltpu.SemaphoreType.DMA,
        ],
    )
    return f(slot, nlls)


def kernel(x, y, y_pos, W_cluster, logits):
    x2 = _pack_x(x)
    slot, perm, tcol, bc = _route(y_pos, y)
    xs2 = _gather_rows(x2, perm)
    nlls = _tail_nll(bc, xs2, logits, W_cluster, tcol)
    return _finalize(slot, nlls.reshape(NPAD))
